# Initial kernel scaffold; baseline (speedup 1.0000x reference)
import jax
import jax.numpy as jnp
from jax.experimental import pallas as pl


def kernel(x, weights, indices, dw0, pw0, g0, b0, dw1, pw1, g1, b1, w2, g2, b2):
    raise NotImplementedError("write your pallas kernel here")


# speedup vs baseline: 1.0921x; 1.0921x over previous
import functools

import jax
import jax.numpy as jnp
from jax.experimental import pallas as pl
from jax.experimental.pallas import tpu as pltpu

_B, _CIN, _COUT, _H, _W = 32, 384, 384, 32, 32
_CS = _CIN // 2
_HW = _H * _W
_NEXP = 4


def _shift_flat(x, off, w_mask_pos, w_mask_neg, dj):
    c = x.shape[0]
    if off > 0:
        sh = jnp.concatenate([x[:, off:], jnp.zeros((c, off), jnp.float32)], axis=1)
    elif off < 0:
        sh = jnp.concatenate([jnp.zeros((c, -off), jnp.float32), x[:, :off]], axis=1)
    else:
        sh = x
    if dj == 1:
        sh = sh * w_mask_pos
    elif dj == -1:
        sh = sh * w_mask_neg
    return sh


def _dwconv(xh, dw_ref, w_mask_pos, w_mask_neg):
    acc = xh * dw_ref[:, 4:5]
    for di in (-1, 0, 1):
        for dj in (-1, 0, 1):
            if di == 0 and dj == 0:
                continue
            t = (di + 1) * 3 + (dj + 1)
            off = di * _W + dj
            sh = _shift_flat(xh, off, w_mask_pos, w_mask_neg, dj)
            acc = acc + sh * dw_ref[:, t : t + 1]
    return acc


def _silu(z):
    return z * jax.nn.sigmoid(z)


def _tc_body(c_ref, x_ref, dw0_ref, pw0_ref, b0_ref, dw1_ref, pw1_ref, b1_ref,
             w2_ref, b2_ref, out_ref):
    b = pl.program_id(0)
    j = jax.lax.broadcasted_iota(jnp.int32, (1, _HW), 1) % _W
    w_mask_pos = (j != (_W - 1)).astype(jnp.float32)
    w_mask_neg = (j != 0).astype(jnp.float32)

    out_ref[0] = jnp.zeros((_COUT, _HW), jnp.float32)

    c0 = c_ref[b, 0]
    c1 = c_ref[b, 1]
    c2 = c_ref[b, 2]

    @pl.when(c0 != 0.0)
    def _():
        y = _dwconv(x_ref[0, :_CS], dw0_ref, w_mask_pos, w_mask_neg)
        z = jnp.dot(pw0_ref[...], y.astype(jnp.bfloat16),
                    preferred_element_type=jnp.float32) + b0_ref[...]
        out_ref[0] += c0 * _silu(z)

    @pl.when(c1 != 0.0)
    def _():
        y = _dwconv(x_ref[0, _CS:], dw1_ref, w_mask_pos, w_mask_neg)
        z = jnp.dot(pw1_ref[...], y.astype(jnp.bfloat16),
                    preferred_element_type=jnp.float32) + b1_ref[...]
        out_ref[0] += c1 * _silu(z)

    @pl.when(c2 != 0.0)
    def _():
        z = jnp.dot(w2_ref[...], x_ref[0].astype(jnp.bfloat16),
                    preferred_element_type=jnp.float32) + b2_ref[...]
        out_ref[0] += c2 * _silu(z)


@functools.partial(jax.jit, static_argnames=("interpret",))
def _run(x, coeffs, dw0f, pw0f, b0c, dw1f, pw1f, b1c, w2f, b2c, interpret=False):
    xf = x.reshape(_B, _CIN, _HW)
    const = lambda b: (0, 0)
    out = pl.pallas_call(
        _tc_body,
        grid=(_B,),
        in_specs=[
            pl.BlockSpec(memory_space=pltpu.SMEM),
            pl.BlockSpec((1, _CIN, _HW), lambda b: (b, 0, 0)),
            pl.BlockSpec((_CS, 9), const),
            pl.BlockSpec((_COUT, _CS), const),
            pl.BlockSpec((_COUT, 1), const),
            pl.BlockSpec((_CS, 9), const),
            pl.BlockSpec((_COUT, _CS), const),
            pl.BlockSpec((_COUT, 1), const),
            pl.BlockSpec((_COUT, _CIN), const),
            pl.BlockSpec((_COUT, 1), const),
        ],
        out_specs=pl.BlockSpec((1, _COUT, _HW), lambda b: (b, 0, 0)),
        out_shape=jax.ShapeDtypeStruct((_B, _COUT, _HW), jnp.float32),
        interpret=interpret,
    )(coeffs, xf, dw0f, pw0f, b0c, dw1f, pw1f, b1c, w2f, b2c)
    return out.reshape(_B, _COUT, _H, _W)


def _routing_coeffs(weights, indices):
    c = jnp.zeros((_B, _NEXP), jnp.float32)
    c = c.at[jnp.arange(_B)[:, None], indices].add(weights)
    return c


def kernel(x, weights, indices, dw0, pw0, g0, b0, dw1, pw1, g1, b1, w2, g2, b2):
    eps = 1e-5
    s0 = g0 / jnp.sqrt(1.0 + eps)
    s1 = g1 / jnp.sqrt(1.0 + eps)
    s2 = g2 / jnp.sqrt(1.0 + eps)
    dw0f = dw0.reshape(_CS, 9)
    dw1f = dw1.reshape(_CS, 9)
    pw0f = (pw0.reshape(_COUT, _CS) * s0[:, None]).astype(jnp.bfloat16)
    pw1f = (pw1.reshape(_COUT, _CS) * s1[:, None]).astype(jnp.bfloat16)
    w2f = (w2.reshape(_COUT, _CIN) * s2[:, None]).astype(jnp.bfloat16)
    b0c = b0[:, None]
    b1c = b1[:, None]
    b2c = b2[:, None]
    coeffs = _routing_coeffs(weights, indices)
    return _run(x, coeffs, dw0f, pw0f, b0c, dw1f, pw1f, b1c, w2f, b2c)


# speedup vs baseline: 1.2562x; 1.1502x over previous
import functools

import jax
import jax.numpy as jnp
from jax.experimental import pallas as pl
from jax.experimental.pallas import tpu as pltpu

_B, _CIN, _COUT, _H, _W = 32, 384, 384, 32, 32
_CS = _CIN // 2
_HW = _H * _W
_NEXP = 4


def _shift_flat(x, off):
    c = x.shape[0]
    if off > 0:
        return jnp.concatenate([x[:, off:], jnp.zeros((c, off), x.dtype)], axis=1)
    if off < 0:
        return jnp.concatenate([jnp.zeros((c, -off), x.dtype), x[:, :off]], axis=1)
    return x


def _dwconv(xh, dw_ref, w_mask_pos, w_mask_neg):
    s_neg = _shift_flat(xh, -1) * w_mask_neg
    s_pos = _shift_flat(xh, 1) * w_mask_pos
    v = []
    for di in (-1, 0, 1):
        t = (di + 1) * 3 + 1
        v.append(s_neg * dw_ref[:, t - 1 : t] + xh * dw_ref[:, t : t + 1]
                 + s_pos * dw_ref[:, t + 1 : t + 2])
    return v[1] + _shift_flat(v[0], -_W) + _shift_flat(v[2], _W)


def _silu(z):
    return z * jax.nn.sigmoid(z)


def _tc_body(c_ref, x_ref, dw0_ref, pw0_ref, b0_ref, dw1_ref, pw1_ref, b1_ref,
             w2_ref, b2_ref, out_ref):
    b = pl.program_id(0)
    j = jax.lax.broadcasted_iota(jnp.int32, (1, _HW), 1) % _W
    w_mask_pos = (j != (_W - 1)).astype(jnp.float32)
    w_mask_neg = (j != 0).astype(jnp.float32)

    out_ref[0] = jnp.zeros((_COUT, _HW), jnp.float32)

    c0 = c_ref[b, 0]
    c1 = c_ref[b, 1]
    c2 = c_ref[b, 2]

    @pl.when(c0 != 0.0)
    def _():
        y = _dwconv(x_ref[0, :_CS], dw0_ref, w_mask_pos, w_mask_neg)
        z = jnp.dot(pw0_ref[...], y.astype(jnp.bfloat16),
                    preferred_element_type=jnp.float32) + b0_ref[...]
        out_ref[0] += c0 * _silu(z)

    @pl.when(c1 != 0.0)
    def _():
        y = _dwconv(x_ref[0, _CS:], dw1_ref, w_mask_pos, w_mask_neg)
        z = jnp.dot(pw1_ref[...], y.astype(jnp.bfloat16),
                    preferred_element_type=jnp.float32) + b1_ref[...]
        out_ref[0] += c1 * _silu(z)

    @pl.when(c2 != 0.0)
    def _():
        z = jnp.dot(w2_ref[...], x_ref[0].astype(jnp.bfloat16),
                    preferred_element_type=jnp.float32) + b2_ref[...]
        out_ref[0] += c2 * _silu(z)


@functools.partial(jax.jit, static_argnames=("interpret",))
def _run(x, coeffs, dw0f, pw0f, b0c, dw1f, pw1f, b1c, w2f, b2c, interpret=False):
    xf = x.reshape(_B, _CIN, _HW)
    const = lambda b: (0, 0)
    out = pl.pallas_call(
        _tc_body,
        grid=(_B,),
        in_specs=[
            pl.BlockSpec(memory_space=pltpu.SMEM),
            pl.BlockSpec((1, _CIN, _HW), lambda b: (b, 0, 0)),
            pl.BlockSpec((_CS, 9), const),
            pl.BlockSpec((_COUT, _CS), const),
            pl.BlockSpec((_COUT, 1), const),
            pl.BlockSpec((_CS, 9), const),
            pl.BlockSpec((_COUT, _CS), const),
            pl.BlockSpec((_COUT, 1), const),
            pl.BlockSpec((_COUT, _CIN), const),
            pl.BlockSpec((_COUT, 1), const),
        ],
        out_specs=pl.BlockSpec((1, _COUT, _HW), lambda b: (b, 0, 0)),
        out_shape=jax.ShapeDtypeStruct((_B, _COUT, _HW), jnp.float32),
        interpret=interpret,
    )(coeffs, xf, dw0f, pw0f, b0c, dw1f, pw1f, b1c, w2f, b2c)
    return out.reshape(_B, _COUT, _H, _W)


def _routing_coeffs(weights, indices):
    c = jnp.zeros((_B, _NEXP), jnp.float32)
    c = c.at[jnp.arange(_B)[:, None], indices].add(weights)
    return c


def kernel(x, weights, indices, dw0, pw0, g0, b0, dw1, pw1, g1, b1, w2, g2, b2):
    eps = 1e-5
    s0 = g0 / jnp.sqrt(1.0 + eps)
    s1 = g1 / jnp.sqrt(1.0 + eps)
    s2 = g2 / jnp.sqrt(1.0 + eps)
    dw0f = dw0.reshape(_CS, 9)
    dw1f = dw1.reshape(_CS, 9)
    pw0f = (pw0.reshape(_COUT, _CS) * s0[:, None]).astype(jnp.bfloat16)
    pw1f = (pw1.reshape(_COUT, _CS) * s1[:, None]).astype(jnp.bfloat16)
    w2f = (w2.reshape(_COUT, _CIN) * s2[:, None]).astype(jnp.bfloat16)
    b0c = b0[:, None]
    b1c = b1[:, None]
    b2c = b2[:, None]
    coeffs = _routing_coeffs(weights, indices)
    return _run(x, coeffs, dw0f, pw0f, b0c, dw1f, pw1f, b1c, w2f, b2c)


# speedup vs baseline: 1.4152x; 1.1266x over previous
import functools

import jax
import jax.numpy as jnp
from jax.experimental import pallas as pl
from jax.experimental.pallas import tpu as pltpu

_B, _CIN, _COUT, _H, _W = 32, 384, 384, 32, 32
_CS = _CIN // 2
_HW = _H * _W
_NEXP = 4


def _shift_flat(x, off):
    c = x.shape[0]
    if off > 0:
        return jnp.concatenate([x[:, off:], jnp.zeros((c, off), x.dtype)], axis=1)
    if off < 0:
        return jnp.concatenate([jnp.zeros((c, -off), x.dtype), x[:, :off]], axis=1)
    return x


def _dwconv(xh, dw_ref, w_mask_pos, w_mask_neg):
    s_neg = _shift_flat(xh, -1) * w_mask_neg
    s_pos = _shift_flat(xh, 1) * w_mask_pos
    v = []
    for di in (-1, 0, 1):
        t = (di + 1) * 3 + 1
        v.append(s_neg * dw_ref[:, t - 1 : t] + xh * dw_ref[:, t : t + 1]
                 + s_pos * dw_ref[:, t + 1 : t + 2])
    return v[1] + _shift_flat(v[0], -_W) + _shift_flat(v[2], _W)


def _silu(z):
    return z * jax.nn.sigmoid(z)


def _tc_body(wt_ref, idx_ref, x_ref, dw0_ref, pw0_ref, b0_ref, dw1_ref, pw1_ref,
             b1_ref, w2_ref, b2_ref, out_ref):
    b = pl.program_id(0)
    j = jax.lax.broadcasted_iota(jnp.int32, (1, _HW), 1) % _W
    w_mask_pos = (j != (_W - 1)).astype(jnp.float32)
    w_mask_neg = (j != 0).astype(jnp.float32)

    out_ref[0] = jnp.zeros((_COUT, _HW), jnp.float32)

    i0 = idx_ref[b, 0]
    i1 = idx_ref[b, 1]
    w0 = wt_ref[b, 0]
    w1 = wt_ref[b, 1]
    zero = jnp.float32(0.0)
    c0 = jnp.where(i0 == 0, w0, zero) + jnp.where(i1 == 0, w1, zero)
    c1 = jnp.where(i0 == 1, w0, zero) + jnp.where(i1 == 1, w1, zero)
    c2 = jnp.where(i0 == 2, w0, zero) + jnp.where(i1 == 2, w1, zero)

    @pl.when(c0 != 0.0)
    def _():
        y = _dwconv(x_ref[0, :_CS], dw0_ref, w_mask_pos, w_mask_neg)
        z = jnp.dot(pw0_ref[...], y.astype(jnp.bfloat16),
                    preferred_element_type=jnp.float32) + b0_ref[...]
        out_ref[0] += c0 * _silu(z)

    @pl.when(c1 != 0.0)
    def _():
        y = _dwconv(x_ref[0, _CS:], dw1_ref, w_mask_pos, w_mask_neg)
        z = jnp.dot(pw1_ref[...], y.astype(jnp.bfloat16),
                    preferred_element_type=jnp.float32) + b1_ref[...]
        out_ref[0] += c1 * _silu(z)

    @pl.when(c2 != 0.0)
    def _():
        z = jnp.dot(w2_ref[...], x_ref[0].astype(jnp.bfloat16),
                    preferred_element_type=jnp.float32) + b2_ref[...]
        out_ref[0] += c2 * _silu(z)


@functools.partial(jax.jit, static_argnames=("interpret",))
def _run(x, wts, idxs, dw0f, pw0f, b0c, dw1f, pw1f, b1c, w2f, b2c, interpret=False):
    xf = x.reshape(_B, _CIN, _HW)
    const = lambda b: (0, 0)
    out = pl.pallas_call(
        _tc_body,
        grid=(_B,),
        in_specs=[
            pl.BlockSpec(memory_space=pltpu.SMEM),
            pl.BlockSpec(memory_space=pltpu.SMEM),
            pl.BlockSpec((1, _CIN, _HW), lambda b: (b, 0, 0)),
            pl.BlockSpec((_CS, 9), const),
            pl.BlockSpec((_COUT, _CS), const),
            pl.BlockSpec((_COUT, 1), const),
            pl.BlockSpec((_CS, 9), const),
            pl.BlockSpec((_COUT, _CS), const),
            pl.BlockSpec((_COUT, 1), const),
            pl.BlockSpec((_COUT, _CIN), const),
            pl.BlockSpec((_COUT, 1), const),
        ],
        out_specs=pl.BlockSpec((1, _COUT, _HW), lambda b: (b, 0, 0)),
        out_shape=jax.ShapeDtypeStruct((_B, _COUT, _HW), jnp.float32),
        interpret=interpret,
    )(wts, idxs, xf, dw0f, pw0f, b0c, dw1f, pw1f, b1c, w2f, b2c)
    return out.reshape(_B, _COUT, _H, _W)


def kernel(x, weights, indices, dw0, pw0, g0, b0, dw1, pw1, g1, b1, w2, g2, b2):
    eps = 1e-5
    s0 = g0 / jnp.sqrt(1.0 + eps)
    s1 = g1 / jnp.sqrt(1.0 + eps)
    s2 = g2 / jnp.sqrt(1.0 + eps)
    dw0f = dw0.reshape(_CS, 9)
    dw1f = dw1.reshape(_CS, 9)
    pw0f = (pw0.reshape(_COUT, _CS) * s0[:, None]).astype(jnp.bfloat16)
    pw1f = (pw1.reshape(_COUT, _CS) * s1[:, None]).astype(jnp.bfloat16)
    w2f = (w2.reshape(_COUT, _CIN) * s2[:, None]).astype(jnp.bfloat16)
    b0c = b0[:, None]
    b1c = b1[:, None]
    b2c = b2[:, None]
    return _run(x, weights, indices, dw0f, pw0f, b0c, dw1f, pw1f, b1c, w2f, b2c)


# speedup vs baseline: 1.4432x; 1.0198x over previous
import functools

import jax
import jax.numpy as jnp
from jax.experimental import pallas as pl
from jax.experimental.pallas import tpu as pltpu

_B, _CIN, _COUT, _H, _W = 32, 384, 384, 32, 32
_CS = _CIN // 2
_HW = _H * _W
_NEXP = 4


def _shift_flat(x, off):
    c = x.shape[0]
    if off > 0:
        return jnp.concatenate([x[:, off:], jnp.zeros((c, off), x.dtype)], axis=1)
    if off < 0:
        return jnp.concatenate([jnp.zeros((c, -off), x.dtype), x[:, :off]], axis=1)
    return x


def _dwconv(xh, dw_ref, w_mask_pos, w_mask_neg):
    s_neg = _shift_flat(xh, -1) * w_mask_neg
    s_pos = _shift_flat(xh, 1) * w_mask_pos
    v = []
    for di in (-1, 0, 1):
        t = (di + 1) * 3 + 1
        v.append(s_neg * dw_ref[:, t - 1 : t] + xh * dw_ref[:, t : t + 1]
                 + s_pos * dw_ref[:, t + 1 : t + 2])
    return v[1] + _shift_flat(v[0], -_W) + _shift_flat(v[2], _W)


def _silu(z):
    return z * jax.nn.sigmoid(z)


def _tc_body(wt_ref, idx_ref, x_ref, dw0_ref, pw0_ref, b0_ref, dw1_ref, pw1_ref,
             b1_ref, w2_ref, b2_ref, out_ref):
    b = pl.program_id(0)
    j = jax.lax.broadcasted_iota(jnp.int32, (1, _HW), 1) % _W
    w_mask_pos = (j != (_W - 1)).astype(jnp.bfloat16)
    w_mask_neg = (j != 0).astype(jnp.bfloat16)

    out_ref[0] = jnp.zeros((_COUT, _HW), jnp.float32)

    i0 = idx_ref[b, 0]
    i1 = idx_ref[b, 1]
    w0 = wt_ref[b, 0]
    w1 = wt_ref[b, 1]
    zero = jnp.float32(0.0)
    c0 = jnp.where(i0 == 0, w0, zero) + jnp.where(i1 == 0, w1, zero)
    c1 = jnp.where(i0 == 1, w0, zero) + jnp.where(i1 == 1, w1, zero)
    c2 = jnp.where(i0 == 2, w0, zero) + jnp.where(i1 == 2, w1, zero)

    @pl.when(c0 != 0.0)
    def _():
        y = _dwconv(x_ref[0, :_CS], dw0_ref, w_mask_pos, w_mask_neg)
        z = jnp.dot(pw0_ref[...], y,
                    preferred_element_type=jnp.float32) + b0_ref[...]
        out_ref[0] += c0 * _silu(z)

    @pl.when(c1 != 0.0)
    def _():
        y = _dwconv(x_ref[0, _CS:], dw1_ref, w_mask_pos, w_mask_neg)
        z = jnp.dot(pw1_ref[...], y,
                    preferred_element_type=jnp.float32) + b1_ref[...]
        out_ref[0] += c1 * _silu(z)

    @pl.when(c2 != 0.0)
    def _():
        z = jnp.dot(w2_ref[...], x_ref[0],
                    preferred_element_type=jnp.float32) + b2_ref[...]
        out_ref[0] += c2 * _silu(z)


@functools.partial(jax.jit, static_argnames=("interpret",))
def _run(x, wts, idxs, dw0f, pw0f, b0c, dw1f, pw1f, b1c, w2f, b2c, interpret=False):
    xf = x.reshape(_B, _CIN, _HW).astype(jnp.bfloat16)
    const = lambda b: (0, 0)
    out = pl.pallas_call(
        _tc_body,
        grid=(_B,),
        in_specs=[
            pl.BlockSpec(memory_space=pltpu.SMEM),
            pl.BlockSpec(memory_space=pltpu.SMEM),
            pl.BlockSpec((1, _CIN, _HW), lambda b: (b, 0, 0)),
            pl.BlockSpec((_CS, 9), const),
            pl.BlockSpec((_COUT, _CS), const),
            pl.BlockSpec((_COUT, 1), const),
            pl.BlockSpec((_CS, 9), const),
            pl.BlockSpec((_COUT, _CS), const),
            pl.BlockSpec((_COUT, 1), const),
            pl.BlockSpec((_COUT, _CIN), const),
            pl.BlockSpec((_COUT, 1), const),
        ],
        out_specs=pl.BlockSpec((1, _COUT, _HW), lambda b: (b, 0, 0)),
        out_shape=jax.ShapeDtypeStruct((_B, _COUT, _HW), jnp.float32),
        interpret=interpret,
    )(wts, idxs, xf, dw0f, pw0f, b0c, dw1f, pw1f, b1c, w2f, b2c)
    return out.reshape(_B, _COUT, _H, _W)


def kernel(x, weights, indices, dw0, pw0, g0, b0, dw1, pw1, g1, b1, w2, g2, b2):
    eps = 1e-5
    s0 = g0 / jnp.sqrt(1.0 + eps)
    s1 = g1 / jnp.sqrt(1.0 + eps)
    s2 = g2 / jnp.sqrt(1.0 + eps)
    dw0f = dw0.reshape(_CS, 9).astype(jnp.bfloat16)
    dw1f = dw1.reshape(_CS, 9).astype(jnp.bfloat16)
    pw0f = (pw0.reshape(_COUT, _CS) * s0[:, None]).astype(jnp.bfloat16)
    pw1f = (pw1.reshape(_COUT, _CS) * s1[:, None]).astype(jnp.bfloat16)
    w2f = (w2.reshape(_COUT, _CIN) * s2[:, None]).astype(jnp.bfloat16)
    b0c = b0[:, None]
    b1c = b1[:, None]
    b2c = b2[:, None]
    return _run(x, weights, indices, dw0f, pw0f, b0c, dw1f, pw1f, b1c, w2f, b2c)


# speedup vs baseline: 1.5159x; 1.0504x over previous
import functools

import jax
import jax.numpy as jnp
from jax.experimental import pallas as pl
from jax.experimental.pallas import tpu as pltpu

_B, _CIN, _COUT, _H, _W = 32, 384, 384, 32, 32
_CS = _CIN // 2
_HW = _H * _W
_NEXP = 4


def _shift_flat(x, off):
    c = x.shape[0]
    if off > 0:
        return jnp.concatenate([x[:, off:], jnp.zeros((c, off), x.dtype)], axis=1)
    if off < 0:
        return jnp.concatenate([jnp.zeros((c, -off), x.dtype), x[:, :off]], axis=1)
    return x


def _dwconv(xh, dw_ref, w_mask_pos, w_mask_neg):
    s_neg = _shift_flat(xh, -1) * w_mask_neg
    s_pos = _shift_flat(xh, 1) * w_mask_pos
    v = []
    for di in (-1, 0, 1):
        t = (di + 1) * 3 + 1
        v.append(s_neg * dw_ref[:, t - 1 : t] + xh * dw_ref[:, t : t + 1]
                 + s_pos * dw_ref[:, t + 1 : t + 2])
    return v[1] + _shift_flat(v[0], -_W) + _shift_flat(v[2], _W)


def _silu(z):
    return z * jax.nn.sigmoid(z)


def _tc_body(wt_ref, idx_ref, x_ref, dw0_ref, pw0_ref, b0_ref, dw1_ref, pw1_ref,
             b1_ref, w2_ref, b2_ref, out_ref):
    b = pl.program_id(0)
    j = jax.lax.broadcasted_iota(jnp.int32, (1, _HW), 1) % _W
    w_mask_pos = (j != (_W - 1)).astype(jnp.bfloat16)
    w_mask_neg = (j != 0).astype(jnp.bfloat16)

    out_ref[0] = jnp.zeros((_COUT, _HW), jnp.float32)

    i0 = idx_ref[b, 0]
    i1 = idx_ref[b, 1]
    w0 = wt_ref[b, 0]
    w1 = wt_ref[b, 1]
    zero = jnp.float32(0.0)
    c0 = jnp.where(i0 == 0, w0, zero) + jnp.where(i1 == 0, w1, zero)
    c1 = jnp.where(i0 == 1, w0, zero) + jnp.where(i1 == 1, w1, zero)
    c2 = jnp.where(i0 == 2, w0, zero) + jnp.where(i1 == 2, w1, zero)

    xb = x_ref[0].astype(jnp.bfloat16)

    @pl.when(c0 != 0.0)
    def _():
        y = _dwconv(xb[:_CS], dw0_ref, w_mask_pos, w_mask_neg)
        z = jnp.dot(pw0_ref[...], y,
                    preferred_element_type=jnp.float32) + b0_ref[...]
        out_ref[0] += c0 * _silu(z)

    @pl.when(c1 != 0.0)
    def _():
        y = _dwconv(xb[_CS:], dw1_ref, w_mask_pos, w_mask_neg)
        z = jnp.dot(pw1_ref[...], y,
                    preferred_element_type=jnp.float32) + b1_ref[...]
        out_ref[0] += c1 * _silu(z)

    @pl.when(c2 != 0.0)
    def _():
        z = jnp.dot(w2_ref[...], xb,
                    preferred_element_type=jnp.float32) + b2_ref[...]
        out_ref[0] += c2 * _silu(z)


@functools.partial(jax.jit, static_argnames=("interpret",))
def _run(x, wts, idxs, dw0f, pw0f, b0c, dw1f, pw1f, b1c, w2f, b2c, interpret=False):
    xf = x.reshape(_B, _CIN, _HW)
    const = lambda b: (0, 0)
    out = pl.pallas_call(
        _tc_body,
        grid=(_B,),
        in_specs=[
            pl.BlockSpec(memory_space=pltpu.SMEM),
            pl.BlockSpec(memory_space=pltpu.SMEM),
            pl.BlockSpec((1, _CIN, _HW), lambda b: (b, 0, 0)),
            pl.BlockSpec((_CS, 9), const),
            pl.BlockSpec((_COUT, _CS), const),
            pl.BlockSpec((_COUT, 1), const),
            pl.BlockSpec((_CS, 9), const),
            pl.BlockSpec((_COUT, _CS), const),
            pl.BlockSpec((_COUT, 1), const),
            pl.BlockSpec((_COUT, _CIN), const),
            pl.BlockSpec((_COUT, 1), const),
        ],
        out_specs=pl.BlockSpec((1, _COUT, _HW), lambda b: (b, 0, 0)),
        out_shape=jax.ShapeDtypeStruct((_B, _COUT, _HW), jnp.float32),
        interpret=interpret,
    )(wts, idxs, xf, dw0f, pw0f, b0c, dw1f, pw1f, b1c, w2f, b2c)
    return out.reshape(_B, _COUT, _H, _W)


def kernel(x, weights, indices, dw0, pw0, g0, b0, dw1, pw1, g1, b1, w2, g2, b2):
    eps = 1e-5
    s0 = g0 / jnp.sqrt(1.0 + eps)
    s1 = g1 / jnp.sqrt(1.0 + eps)
    s2 = g2 / jnp.sqrt(1.0 + eps)
    dw0f = dw0.reshape(_CS, 9).astype(jnp.bfloat16)
    dw1f = dw1.reshape(_CS, 9).astype(jnp.bfloat16)
    pw0f = (pw0.reshape(_COUT, _CS) * s0[:, None]).astype(jnp.bfloat16)
    pw1f = (pw1.reshape(_COUT, _CS) * s1[:, None]).astype(jnp.bfloat16)
    w2f = (w2.reshape(_COUT, _CIN) * s2[:, None]).astype(jnp.bfloat16)
    b0c = b0[:, None]
    b1c = b1[:, None]
    b2c = b2[:, None]
    return _run(x, weights, indices, dw0f, pw0f, b0c, dw1f, pw1f, b1c, w2f, b2c)
